# Initial kernel scaffold; baseline (speedup 1.0000x reference)
import jax
import jax.numpy as jnp
from jax.experimental import pallas as pl


def kernel(x, edge_index, edge_type, W1, root1, b1, W2, root2, b2):
    raise NotImplementedError("write your pallas kernel here")


# speedup vs baseline: 16.6188x; 16.6188x over previous
import functools

import jax
import jax.numpy as jnp
from jax import lax
from jax.experimental import pallas as pl
from jax.experimental.pallas import tpu as pltpu
from jax.experimental.pallas import tpu_sc as plsc

N = 10000
E = 320000
F = 128
R = 16
J = R + 1

NC = 2
NS = 16
L = 16
NW = NC * NS

C = 256
SUB = 128
NSUB = C // SUB
NCH = 40
EP = NW * NCH * C
NP = 10240
ROWS_PT = NP // NS
NB = 400
UE = 8

_f32 = jnp.float32
_i32 = jnp.int32


def _mesh():
  return plsc.VectorSubcoreMesh(
      core_axis_name="c", subcore_axis_name="s", num_cores=NC, num_subcores=NS)


ZR = 2048


def _counts_body(dst_hbm, typ_hbm, cnt_hbm, dst_v, typ_v, fidx_v, ones_v,
                 zst_v, acc, sem):
  cid = lax.axis_index("c")
  sid = lax.axis_index("s")
  wid = sid * NC + cid
  fbase = sid * (ROWS_PT * L)

  zeros = jnp.zeros((L,), _f32)
  for g in range(SUB // L):
    ones_v[pl.ds(g * L, L)] = jnp.ones((L,), _f32)
  for g in range(ZR // L):
    zst_v[pl.ds(g * L, L)] = zeros
  for k in range(ROWS_PT * L // ZR):
    pltpu.sync_copy(zst_v, acc.at[pl.ds(fbase + k * ZR, ZR)])
  plsc.subcore_barrier()

  ebase = wid * (NCH * C)

  def chunk(ch, carry):
    base = ebase + ch * C
    pltpu.sync_copy(dst_hbm.at[pl.ds(base, C)], dst_v)
    pltpu.sync_copy(typ_hbm.at[pl.ds(base, C)], typ_v)
    cps = []
    for j in range(NSUB):
      for g in range(SUB // L):
        off = j * SUB + g * L
        dv = dst_v[pl.ds(off, L)]
        tv = typ_v[pl.ds(off, L)]
        fidx_v[j, pl.ds(g * L, L)] = (dv << 4) + tv
      cps.append(pltpu.async_copy(ones_v, acc.at[fidx_v.at[j]], sem, add=True))
    for cp in cps:
      cp.wait()
    return carry

  lax.fori_loop(0, NCH, chunk, 0)
  plsc.subcore_barrier()
  pltpu.sync_copy(acc.at[pl.ds(fbase, ROWS_PT * L)],
                  cnt_hbm.at[cid, pl.ds(fbase, ROWS_PT * L)])


_SC_PARAMS = pltpu.CompilerParams(needs_layout_passes=False)

_counts_call = functools.partial(
    pl.kernel,
    out_type=jax.ShapeDtypeStruct((NC, NP * L), _f32),
    compiler_params=_SC_PARAMS,
    scratch_types=[
        pltpu.VMEM((C,), _i32),
        pltpu.VMEM((C,), _i32),
        pltpu.VMEM((NSUB, SUB), _i32),
        pltpu.VMEM((SUB,), _f32),
        pltpu.VMEM((ZR,), _f32),
        pltpu.VMEM_SHARED((NP * L,), _f32),
        pltpu.SemaphoreType.DMA,
    ],
)


def _weights_body(cnt2_hbm, dst_hbm, typ_hbm, w_hbm,
                  dst_v, typ_v, fidx_v, fidxB_v, c0_v, c1_v, w_v, sem):
  cid = lax.axis_index("c")
  sid = lax.axis_index("s")
  wid = sid * NC + cid
  ebase = wid * (NCH * C)

  def chunk(ch, carry):
    base = ebase + ch * C
    pltpu.sync_copy(dst_hbm.at[pl.ds(base, C)], dst_v)
    pltpu.sync_copy(typ_hbm.at[pl.ds(base, C)], typ_v)
    cps = []
    for j in range(NSUB):
      for g in range(SUB // L):
        off = j * SUB + g * L
        dv = dst_v[pl.ds(off, L)]
        tv = typ_v[pl.ds(off, L)]
        fi = (dv << 4) + tv
        fidx_v[j, pl.ds(g * L, L)] = fi
        fidxB_v[j, pl.ds(g * L, L)] = fi + NP * L
      cps.append(pltpu.async_copy(
          cnt2_hbm.at[fidx_v.at[j]], c0_v.at[pl.ds(j * SUB, SUB)], sem))
      cps.append(pltpu.async_copy(
          cnt2_hbm.at[fidxB_v.at[j]], c1_v.at[pl.ds(j * SUB, SUB)], sem))
    for cp in cps:
      cp.wait()
    for g in range(C // L):
      off = g * L
      s0 = c0_v[pl.ds(off, L)]
      s1 = c1_v[pl.ds(off, L)]
      w_v[pl.ds(off, L)] = 1.0 / jnp.maximum(s0 + s1, 1.0)
    pltpu.sync_copy(w_v, w_hbm.at[pl.ds(base, C)])
    return carry

  lax.fori_loop(0, NCH, chunk, 0)


_weights_call = functools.partial(
    pl.kernel,
    out_type=jax.ShapeDtypeStruct((EP,), _f32),
    compiler_params=_SC_PARAMS,
    scratch_types=[
        pltpu.VMEM((C,), _i32),
        pltpu.VMEM((C,), _i32),
        pltpu.VMEM((NSUB, SUB), _i32),
        pltpu.VMEM((NSUB, SUB), _i32),
        pltpu.VMEM((C,), _f32),
        pltpu.VMEM((C,), _f32),
        pltpu.VMEM((C,), _f32),
        pltpu.SemaphoreType.DMA,
    ],
)


def _edges_body(tab_hbm, src_hbm, typ_hbm, dst_hbm, w_hbm, z_hbm, acc_hbm,
                s_v, t_v, g_v, w_v, d2_v, rows_v, acc, sem, sem2):
  cid = lax.axis_index("c")
  sid = lax.axis_index("s")
  wid = sid * NC + cid
  row0 = sid * ROWS_PT

  pltpu.sync_copy(z_hbm.at[pl.ds(row0, ROWS_PT)], acc.at[pl.ds(row0, ROWS_PT)])
  plsc.subcore_barrier()

  ebase = wid * (NCH * C)

  def chunk(ch, carry):
    base = ebase + ch * C
    cps = [
        pltpu.async_copy(src_hbm.at[pl.ds(base, C)], s_v, sem),
        pltpu.async_copy(typ_hbm.at[pl.ds(base, C)], t_v, sem),
        pltpu.async_copy(w_hbm.at[pl.ds(base, C)], w_v, sem),
    ]
    for j in range(NSUB):
      cps.append(pltpu.async_copy(
          dst_hbm.at[pl.ds(base + j * SUB, SUB)], d2_v.at[j], sem))
    for cp in cps:
      cp.wait()
    for g in range(C // L):
      sv = s_v[pl.ds(g * L, L)]
      tv = t_v[pl.ds(g * L, L)]
      g_v[pl.ds(g * L, L)] = tv * N + sv
    gcps = [
        pltpu.async_copy(
            tab_hbm.at[g_v.at[pl.ds(j * SUB, SUB)]],
            rows_v.at[pl.ds(j * SUB, SUB)], sem)
        for j in range(NSUB)
    ]
    for cp in gcps:
      cp.wait()
    scps = []
    for j in range(NSUB):
      def sloop(it, carry2, j=j):
        e0 = j * SUB + it * UE
        for u in range(UE):
          e = e0 + u
          ws = plsc.load_gather(w_v, [jnp.full((L,), e, _i32)])
          for k in range(F // L):
            sl = pl.ds(k * L, L)
            rows_v[e, sl] = rows_v[e, sl] * ws
        return carry2
      lax.fori_loop(0, SUB // UE, sloop, 0)
      scps.append(pltpu.async_copy(
          rows_v.at[pl.ds(j * SUB, SUB)], acc.at[d2_v.at[j]], sem2, add=True))
    for cp in scps:
      cp.wait()
    return carry

  lax.fori_loop(0, NCH, chunk, 0)
  plsc.subcore_barrier()
  pltpu.sync_copy(acc.at[pl.ds(row0, ROWS_PT)],
                  acc_hbm.at[cid, pl.ds(row0, ROWS_PT)])


_edges_call = functools.partial(
    pl.kernel,
    out_type=jax.ShapeDtypeStruct((NC, NP, F), _f32),
    compiler_params=_SC_PARAMS,
    scratch_types=[
        pltpu.VMEM((C,), _i32),
        pltpu.VMEM((C,), _i32),
        pltpu.VMEM((C,), _i32),
        pltpu.VMEM((C,), _f32),
        pltpu.VMEM((NSUB, SUB), _i32),
        pltpu.VMEM((C, F), _f32),
        pltpu.VMEM_SHARED((NP, F), _f32),
        pltpu.SemaphoreType.DMA,
        pltpu.SemaphoreType.DMA,
    ],
)


def _mm0_body(x_ref, w_ref, b_ref, o_ref):
  o_ref[0] = (
      jnp.dot(x_ref[...], w_ref[0], preferred_element_type=_f32) + b_ref[0, 0])


def _mm1_body(t1_ref, a_ref, w_ref, b_ref, o_ref):
  h = jnp.maximum(t1_ref[0] + a_ref[0] + a_ref[1], 0.0)
  o_ref[0] = (
      jnp.dot(h, w_ref[0], preferred_element_type=_f32) + b_ref[0, 0])


def _add_body(t2_ref, a_ref, o_ref):
  o_ref[...] = t2_ref[0] + a_ref[0] + a_ref[1]


_mm0 = pl.pallas_call(
    _mm0_body,
    grid=(N // NB, J),
    in_specs=[
        pl.BlockSpec((NB, F), lambda i, j: (i, 0)),
        pl.BlockSpec((1, F, F), lambda i, j: (j, 0, 0)),
        pl.BlockSpec((1, 1, F), lambda i, j: (j, 0, 0)),
    ],
    out_specs=pl.BlockSpec((1, NB, F), lambda i, j: (j, i, 0)),
    out_shape=jax.ShapeDtypeStruct((J, N, F), _f32),
)

_mm1 = pl.pallas_call(
    _mm1_body,
    grid=(N // NB, J),
    in_specs=[
        pl.BlockSpec((1, NB, F), lambda i, j: (R, i, 0)),
        pl.BlockSpec((NC, NB, F), lambda i, j: (0, i, 0)),
        pl.BlockSpec((1, F, F), lambda i, j: (j, 0, 0)),
        pl.BlockSpec((1, 1, F), lambda i, j: (j, 0, 0)),
    ],
    out_specs=pl.BlockSpec((1, NB, F), lambda i, j: (j, i, 0)),
    out_shape=jax.ShapeDtypeStruct((J, N, F), _f32),
)

_addk = pl.pallas_call(
    _add_body,
    grid=(N // NB,),
    in_specs=[
        pl.BlockSpec((1, NB, F), lambda i: (R, i, 0)),
        pl.BlockSpec((NC, NB, F), lambda i: (0, i, 0)),
    ],
    out_specs=pl.BlockSpec((NB, F), lambda i: (i, 0)),
    out_shape=jax.ShapeDtypeStruct((N, F), _f32),
)


def kernel(x, edge_index, edge_type, W1, root1, b1, W2, root2, b2):
  x = x.astype(_f32)
  src = edge_index[0].astype(_i32)
  dst = edge_index[1].astype(_i32)
  typ = edge_type.astype(_i32)
  pad = EP - E
  src_p = jnp.pad(src, (0, pad))
  dst_p = jnp.pad(dst, (0, pad), constant_values=NP - 1)
  typ_p = jnp.pad(typ, (0, pad))

  Waug1 = jnp.concatenate([W1, root1[None]], axis=0)
  baug1 = jnp.concatenate(
      [jnp.zeros((R, F), _f32), b1[None]], axis=0).reshape(J, 1, F)
  Waug2 = jnp.concatenate([W2, root2[None]], axis=0)
  baug2 = jnp.concatenate(
      [jnp.zeros((R, F), _f32), b2[None]], axis=0).reshape(J, 1, F)
  zfull = jnp.zeros((NP, F), _f32)

  mesh = _mesh()
  cnt = _counts_call(_counts_body, mesh=mesh)(dst_p, typ_p)
  w = _weights_call(_weights_body, mesh=mesh)(
      cnt.reshape(NC * NP * L), dst_p, typ_p)

  T1 = _mm0(x, Waug1, baug1)
  acc1 = _edges_call(_edges_body, mesh=mesh)(
      T1.reshape(J * N, F), src_p, typ_p, dst_p, w, zfull)
  T2 = _mm1(T1, acc1, Waug2, baug2)
  acc2 = _edges_call(_edges_body, mesh=mesh)(
      T2.reshape(J * N, F), src_p, typ_p, dst_p, w, zfull)
  return _addk(T2, acc2)


# speedup vs baseline: 24.6844x; 1.4853x over previous
import functools

import jax
import jax.numpy as jnp
from jax import lax
from jax.experimental import pallas as pl
from jax.experimental.pallas import tpu as pltpu
from jax.experimental.pallas import tpu_sc as plsc

N = 10000
E = 320000
F = 128
R = 16
J = R + 1

NC = 2
NS = 16
L = 16
NW = NC * NS

C = 256
SUB = 128
NSUB = C // SUB
NCH = 40
EP = NW * NCH * C
NP = 10240
ROWS_PT = NP // NS
NB = 400
UE = 8
CH_SPLIT = (58, 22)
CH0_CHUNKS = NS * CH_SPLIT[0]

_f32 = jnp.float32
_i32 = jnp.int32


def _mesh():
  return plsc.VectorSubcoreMesh(
      core_axis_name="c", subcore_axis_name="s", num_cores=NC, num_subcores=NS)


ZR = 2048


def _counts_body(dst_hbm, typ_hbm, cnt_hbm, dst_v, typ_v, fidx_v, ones_v,
                 zst_v, acc, sem):
  cid = lax.axis_index("c")
  sid = lax.axis_index("s")
  wid = sid * NC + cid
  fbase = sid * (ROWS_PT * L)

  zeros = jnp.zeros((L,), _f32)
  for g in range(SUB // L):
    ones_v[pl.ds(g * L, L)] = jnp.ones((L,), _f32)
  for g in range(ZR // L):
    zst_v[pl.ds(g * L, L)] = zeros
  for k in range(ROWS_PT * L // ZR):
    pltpu.sync_copy(zst_v, acc.at[pl.ds(fbase + k * ZR, ZR)])
  plsc.subcore_barrier()

  ebase = wid * (NCH * C)

  def chunk(ch, carry):
    base = ebase + ch * C
    pltpu.sync_copy(dst_hbm.at[pl.ds(base, C)], dst_v)
    pltpu.sync_copy(typ_hbm.at[pl.ds(base, C)], typ_v)
    cps = []
    for j in range(NSUB):
      for g in range(SUB // L):
        off = j * SUB + g * L
        dv = dst_v[pl.ds(off, L)]
        tv = typ_v[pl.ds(off, L)]
        fidx_v[j, pl.ds(g * L, L)] = (dv << 4) + tv
      cps.append(pltpu.async_copy(ones_v, acc.at[fidx_v.at[j]], sem, add=True))
    for cp in cps:
      cp.wait()
    return carry

  lax.fori_loop(0, NCH, chunk, 0)
  plsc.subcore_barrier()
  pltpu.sync_copy(acc.at[pl.ds(fbase, ROWS_PT * L)],
                  cnt_hbm.at[cid, pl.ds(fbase, ROWS_PT * L)])


_SC_PARAMS = pltpu.CompilerParams(needs_layout_passes=False)

_counts_call = functools.partial(
    pl.kernel,
    out_type=jax.ShapeDtypeStruct((NC, NP * L), _f32),
    compiler_params=_SC_PARAMS,
    scratch_types=[
        pltpu.VMEM((C,), _i32),
        pltpu.VMEM((C,), _i32),
        pltpu.VMEM((NSUB, SUB), _i32),
        pltpu.VMEM((SUB,), _f32),
        pltpu.VMEM((ZR,), _f32),
        pltpu.VMEM_SHARED((NP * L,), _f32),
        pltpu.SemaphoreType.DMA,
    ],
)


def _weights_body(cnt2_hbm, dst_hbm, typ_hbm, w_hbm,
                  dst_v, typ_v, fidx_v, fidxB_v, c0_v, c1_v, w_v, sem):
  cid = lax.axis_index("c")
  sid = lax.axis_index("s")
  wid = sid * NC + cid
  ebase = wid * (NCH * C)

  def chunk(ch, carry):
    base = ebase + ch * C
    pltpu.sync_copy(dst_hbm.at[pl.ds(base, C)], dst_v)
    pltpu.sync_copy(typ_hbm.at[pl.ds(base, C)], typ_v)
    cps = []
    for j in range(NSUB):
      for g in range(SUB // L):
        off = j * SUB + g * L
        dv = dst_v[pl.ds(off, L)]
        tv = typ_v[pl.ds(off, L)]
        fi = (dv << 4) + tv
        fidx_v[j, pl.ds(g * L, L)] = fi
        fidxB_v[j, pl.ds(g * L, L)] = fi + NP * L
      cps.append(pltpu.async_copy(
          cnt2_hbm.at[fidx_v.at[j]], c0_v.at[pl.ds(j * SUB, SUB)], sem))
      cps.append(pltpu.async_copy(
          cnt2_hbm.at[fidxB_v.at[j]], c1_v.at[pl.ds(j * SUB, SUB)], sem))
    for cp in cps:
      cp.wait()
    for g in range(C // L):
      off = g * L
      s0 = c0_v[pl.ds(off, L)]
      s1 = c1_v[pl.ds(off, L)]
      w_v[pl.ds(off, L)] = 1.0 / jnp.maximum(s0 + s1, 1.0)
    pltpu.sync_copy(w_v, w_hbm.at[pl.ds(base, C)])
    return carry

  lax.fori_loop(0, NCH, chunk, 0)


_weights_call = functools.partial(
    pl.kernel,
    out_type=jax.ShapeDtypeStruct((EP,), _f32),
    compiler_params=_SC_PARAMS,
    scratch_types=[
        pltpu.VMEM((C,), _i32),
        pltpu.VMEM((C,), _i32),
        pltpu.VMEM((NSUB, SUB), _i32),
        pltpu.VMEM((NSUB, SUB), _i32),
        pltpu.VMEM((C,), _f32),
        pltpu.VMEM((C,), _f32),
        pltpu.VMEM((C,), _f32),
        pltpu.SemaphoreType.DMA,
    ],
)


def _edges_body(tab_hbm, src_hbm, typ_hbm, dst_hbm, w_hbm, z_hbm, acc_hbm,
                s_v, t_v, g_v, w_v, d2_v, rows_v, acc, sem, sem2):
  cid = lax.axis_index("c")
  sid = lax.axis_index("s")
  wid = sid * NC + cid
  row0 = sid * ROWS_PT

  pltpu.sync_copy(z_hbm.at[pl.ds(row0, ROWS_PT)], acc.at[pl.ds(row0, ROWS_PT)])
  plsc.subcore_barrier()

  def chunk_body(chunk0, ch):
    base = (chunk0 + ch) * C
    cps = [
        pltpu.async_copy(src_hbm.at[pl.ds(base, C)], s_v, sem),
        pltpu.async_copy(typ_hbm.at[pl.ds(base, C)], t_v, sem),
        pltpu.async_copy(w_hbm.at[pl.ds(base, C)], w_v, sem),
    ]
    for j in range(NSUB):
      cps.append(pltpu.async_copy(
          dst_hbm.at[pl.ds(base + j * SUB, SUB)], d2_v.at[j], sem))
    for cp in cps:
      cp.wait()
    for g in range(C // L):
      sv = s_v[pl.ds(g * L, L)]
      tv = t_v[pl.ds(g * L, L)]
      g_v[pl.ds(g * L, L)] = tv * N + sv
    gcps = [
        pltpu.async_copy(
            tab_hbm.at[g_v.at[pl.ds(j * SUB, SUB)]],
            rows_v.at[pl.ds(j * SUB, SUB)], sem)
        for j in range(NSUB)
    ]
    scps = []
    for j in range(NSUB):
      gcps[j].wait()
      def sloop(it, carry2, j=j):
        e0 = j * SUB + it * UE
        for u in range(UE):
          e = e0 + u
          ws = plsc.load_gather(w_v, [jnp.full((L,), e, _i32)])
          for k in range(F // L):
            sl = pl.ds(k * L, L)
            rows_v[e, sl] = rows_v[e, sl] * ws
        return carry2
      lax.fori_loop(0, SUB // UE, sloop, 0)
      scps.append(pltpu.async_copy(
          rows_v.at[pl.ds(j * SUB, SUB)], acc.at[d2_v.at[j]], sem2, add=True))
    for cp in scps:
      cp.wait()

  @pl.when(cid == 0)
  def _():
    def chunk(ch, carry):
      chunk_body(sid * CH_SPLIT[0], ch)
      return carry
    lax.fori_loop(0, CH_SPLIT[0], chunk, 0)

  @pl.when(cid == 1)
  def _():
    def chunk(ch, carry):
      chunk_body(CH0_CHUNKS + sid * CH_SPLIT[1], ch)
      return carry
    lax.fori_loop(0, CH_SPLIT[1], chunk, 0)
  plsc.subcore_barrier()
  pltpu.sync_copy(acc.at[pl.ds(row0, ROWS_PT)],
                  acc_hbm.at[cid, pl.ds(row0, ROWS_PT)])


_edges_call = functools.partial(
    pl.kernel,
    out_type=jax.ShapeDtypeStruct((NC, NP, F), _f32),
    compiler_params=_SC_PARAMS,
    scratch_types=[
        pltpu.VMEM((C,), _i32),
        pltpu.VMEM((C,), _i32),
        pltpu.VMEM((C,), _i32),
        pltpu.VMEM((C,), _f32),
        pltpu.VMEM((NSUB, SUB), _i32),
        pltpu.VMEM((C, F), _f32),
        pltpu.VMEM_SHARED((NP, F), _f32),
        pltpu.SemaphoreType.DMA,
        pltpu.SemaphoreType.DMA,
    ],
)


_bf16 = jnp.bfloat16


def _mm0_body(x_ref, w_ref, b_ref, o_ref):
  xb = x_ref[...].astype(_bf16)
  for j in range(J):
    o_ref[j] = (
        jnp.dot(xb, w_ref[j].astype(_bf16), preferred_element_type=_f32)
        + b_ref[j, 0])


def _mm1_body(t1_ref, a_ref, w_ref, b_ref, o_ref):
  h = jnp.maximum(t1_ref[0] + a_ref[0] + a_ref[1], 0.0).astype(_bf16)
  for j in range(J):
    o_ref[j] = (
        jnp.dot(h, w_ref[j].astype(_bf16), preferred_element_type=_f32)
        + b_ref[j, 0])


def _add_body(t2_ref, a_ref, o_ref):
  o_ref[...] = t2_ref[0] + a_ref[0] + a_ref[1]


_mm0 = pl.pallas_call(
    _mm0_body,
    grid=(N // NB,),
    in_specs=[
        pl.BlockSpec((NB, F), lambda i: (i, 0)),
        pl.BlockSpec((J, F, F), lambda i: (0, 0, 0)),
        pl.BlockSpec((J, 1, F), lambda i: (0, 0, 0)),
    ],
    out_specs=pl.BlockSpec((J, NB, F), lambda i: (0, i, 0)),
    out_shape=jax.ShapeDtypeStruct((J, N, F), _f32),
)

_mm1 = pl.pallas_call(
    _mm1_body,
    grid=(N // NB,),
    in_specs=[
        pl.BlockSpec((1, NB, F), lambda i: (R, i, 0)),
        pl.BlockSpec((NC, NB, F), lambda i: (0, i, 0)),
        pl.BlockSpec((J, F, F), lambda i: (0, 0, 0)),
        pl.BlockSpec((J, 1, F), lambda i: (0, 0, 0)),
    ],
    out_specs=pl.BlockSpec((J, NB, F), lambda i: (0, i, 0)),
    out_shape=jax.ShapeDtypeStruct((J, N, F), _f32),
)

_addk = pl.pallas_call(
    _add_body,
    grid=(N // NB,),
    in_specs=[
        pl.BlockSpec((1, NB, F), lambda i: (R, i, 0)),
        pl.BlockSpec((NC, NB, F), lambda i: (0, i, 0)),
    ],
    out_specs=pl.BlockSpec((NB, F), lambda i: (i, 0)),
    out_shape=jax.ShapeDtypeStruct((N, F), _f32),
)


def kernel(x, edge_index, edge_type, W1, root1, b1, W2, root2, b2):
  x = x.astype(_f32)
  src = edge_index[0].astype(_i32)
  dst = edge_index[1].astype(_i32)
  typ = edge_type.astype(_i32)
  pad = EP - E
  src_p = jnp.pad(src, (0, pad))
  dst_p = jnp.pad(dst, (0, pad), constant_values=NP - 1)
  typ_p = jnp.pad(typ, (0, pad))

  Waug1 = jnp.concatenate([W1, root1[None]], axis=0)
  baug1 = jnp.concatenate(
      [jnp.zeros((R, F), _f32), b1[None]], axis=0).reshape(J, 1, F)
  Waug2 = jnp.concatenate([W2, root2[None]], axis=0)
  baug2 = jnp.concatenate(
      [jnp.zeros((R, F), _f32), b2[None]], axis=0).reshape(J, 1, F)
  zfull = jnp.zeros((NP, F), _f32)

  mesh = _mesh()
  cnt = _counts_call(_counts_body, mesh=mesh)(dst_p, typ_p)
  w = _weights_call(_weights_body, mesh=mesh)(
      cnt.reshape(NC * NP * L), dst_p, typ_p)

  T1 = _mm0(x, Waug1, baug1)
  acc1 = _edges_call(_edges_body, mesh=mesh)(
      T1.reshape(J * N, F), src_p, typ_p, dst_p, w, zfull)
  T2 = _mm1(T1, acc1, Waug2, baug2)
  acc2 = _edges_call(_edges_body, mesh=mesh)(
      T2.reshape(J * N, F), src_p, typ_p, dst_p, w, zfull)
  return _addk(T2, acc2)
